# Initial kernel scaffold; baseline (speedup 1.0000x reference)
import jax
import jax.numpy as jnp
from jax.experimental import pallas as pl


def kernel(x, edge_index, indices, W1, a1, b1, W2, a2, b2):
    raise NotImplementedError("write your pallas kernel here")


# speedup vs baseline: 8.2522x; 8.2522x over previous
import functools

import jax
import jax.numpy as jnp
from jax.experimental import pallas as pl
from jax.experimental.pallas import tpu as pltpu

F32 = jnp.float32


def _mm_body(x_ref, w_ref, o_ref):
    o_ref[...] = jnp.dot(x_ref[...], w_ref[...], preferred_element_type=F32)


def _matmul(x, W, bn=1000):
    n, k = x.shape
    f = W.shape[1]
    grid = n // bn
    return pl.pallas_call(
        _mm_body,
        grid=(grid,),
        in_specs=[
            pl.BlockSpec((bn, k), lambda i: (i, 0)),
            pl.BlockSpec((k, f), lambda i: (0, 0)),
        ],
        out_specs=pl.BlockSpec((bn, f), lambda i: (i, 0)),
        out_shape=jax.ShapeDtypeStruct((n, f), F32),
    )(x, W)


def _edge_e_body(gs_ref, gd_ref, ab_ref, e_ref):
    m = gs_ref[...] + gd_ref[...]
    m = jnp.where(m >= 0, m, 0.2 * m)
    e_ref[...] = jnp.dot(m, ab_ref[...], preferred_element_type=F32)


def _edge_e(gs, gd, ablock, be=4000):
    e_, f = gs.shape
    h = ablock.shape[1]
    return pl.pallas_call(
        _edge_e_body,
        grid=(e_ // be,),
        in_specs=[
            pl.BlockSpec((be, f), lambda i: (i, 0)),
            pl.BlockSpec((be, f), lambda i: (i, 0)),
            pl.BlockSpec((f, h), lambda i: (0, 0)),
        ],
        out_specs=pl.BlockSpec((be, h), lambda i: (i, 0)),
        out_shape=jax.ShapeDtypeStruct((e_, h), F32),
    )(gs, gd, ablock)


def _alpha_body(e_ref, md_ref, dd_ref, gs_ref, eh_ref, w_ref):
    alpha = jnp.exp(e_ref[...] - md_ref[...]) / dd_ref[...]
    w_ref[...] = jnp.dot(alpha, eh_ref[...],
                         preferred_element_type=F32) * gs_ref[...]


def _alpha_apply(e, md, dd, gs, eh, be=4000):
    e_, h = e.shape
    f = gs.shape[1]
    return pl.pallas_call(
        _alpha_body,
        grid=(e_ // be,),
        in_specs=[
            pl.BlockSpec((be, h), lambda i: (i, 0)),
            pl.BlockSpec((be, h), lambda i: (i, 0)),
            pl.BlockSpec((be, h), lambda i: (i, 0)),
            pl.BlockSpec((be, f), lambda i: (i, 0)),
            pl.BlockSpec((h, f), lambda i: (0, 0)),
        ],
        out_specs=pl.BlockSpec((be, f), lambda i: (i, 0)),
        out_shape=jax.ShapeDtypeStruct((e_, f), F32),
    )(e, md, dd, gs, eh)


def _gat_layer(h, src, dst, ablock, eh, n):
    gs = h[src]
    gd = h[dst]
    e = _edge_e(gs, gd, ablock)
    emax = jax.ops.segment_max(e, dst, num_segments=n)
    emax = jnp.where(jnp.isfinite(emax), emax, 0.0)
    ee = jnp.exp(e - emax[dst])
    denom = jax.ops.segment_sum(ee, dst, num_segments=n) + 1e-9
    w = _alpha_apply(e, emax[dst], denom[dst], gs, eh)
    return jax.ops.segment_sum(w, dst, num_segments=n)


def kernel(x, edge_index, indices, W1, a1, b1, W2, a2, b2):
    n = x.shape[0]
    h1c, u1 = a1.shape
    h2c, u2 = a2.shape
    src = edge_index[0]
    dst = edge_index[1]

    eye1 = jnp.eye(h1c, dtype=F32)
    ab1 = (eye1[:, None, :] * a1[:, :, None]).reshape(h1c * u1, h1c)
    eh1 = jnp.repeat(eye1, u1, axis=1)
    eye2 = jnp.eye(h2c, dtype=F32)
    ab2 = (eye2[:, None, :] * a2[:, :, None]).reshape(h2c * u2, h2c)
    eh2 = jnp.repeat(eye2, u2, axis=1)
    mavg = jnp.tile(jnp.eye(u2, dtype=F32), (h2c, 1)) / h2c

    h1 = _matmul(x, W1)
    s1 = _gat_layer(h1, src, dst, ab1, eh1, n)
    z1 = s1 + b1
    h1b = jnp.where(z1 > 0, z1, jnp.expm1(z1))

    h2 = _matmul(h1b, W2)
    s2 = _gat_layer(h2, src, dst, ab2, eh2, n)
    out = _matmul(s2, mavg) + b2
    return out[indices]


# speedup vs baseline: 29.7720x; 3.6078x over previous
import functools

import jax
import jax.numpy as jnp
from jax import lax
from jax.experimental import pallas as pl
from jax.experimental.pallas import tpu as pltpu, tpu_sc as plsc

F32 = jnp.float32
NC, NS = 2, 16
NW = NC * NS
NP = 10240
CH = 400

_SC_PARAMS = pltpu.CompilerParams(use_tc_tiling_on_sc=False)
_SC_PARAMS_NL = pltpu.CompilerParams(
    use_tc_tiling_on_sc=False, needs_layout_passes=False)
_MESH = plsc.VectorSubcoreMesh(core_axis_name="c", subcore_axis_name="s")


def _mm_body(x_ref, w_ref, o_ref):
    o_ref[...] = jnp.dot(x_ref[...], w_ref[...], preferred_element_type=F32)


def _matmul(x, W, bn=1000):
    n, k = x.shape
    f = W.shape[1]
    return pl.pallas_call(
        _mm_body,
        grid=(n // bn,),
        in_specs=[
            pl.BlockSpec((bn, k), lambda i: (i, 0)),
            pl.BlockSpec((k, f), lambda i: (0, 0)),
        ],
        out_specs=pl.BlockSpec((bn, f), lambda i: (i, 0)),
        out_shape=jax.ShapeDtypeStruct((n, f), F32),
    )(x, W)


def _edge_e_body(gs_ref, gd_ref, ab_ref, e_ref):
    m = gs_ref[...] + gd_ref[...]
    m = jnp.where(m >= 0, m, 0.2 * m)
    e_ref[...] = lax.dot_general(ab_ref[...], m, (((0,), (1,)), ((), ())),
                                 preferred_element_type=F32)


def _edge_eT(gs, gd, ablock, be=6400):
    e_, f = gs.shape
    h = ablock.shape[1]
    return pl.pallas_call(
        _edge_e_body,
        grid=(e_ // be,),
        in_specs=[
            pl.BlockSpec((be, f), lambda i: (i, 0)),
            pl.BlockSpec((be, f), lambda i: (i, 0)),
            pl.BlockSpec((f, h), lambda i: (0, 0)),
        ],
        out_specs=pl.BlockSpec((h, be), lambda i: (0, i)),
        out_shape=jax.ShapeDtypeStruct((h, e_), F32),
    )(gs, gd, ablock)


def _apply1_body(ee_ref, gs_ref, eh_ref, w_ref):
    w_ref[...] = jnp.dot(ee_ref[...], eh_ref[...],
                         preferred_element_type=F32) * gs_ref[...]


def _apply1(ee, gs, eh, be=4000):
    e_, h = ee.shape
    f = gs.shape[1]
    return pl.pallas_call(
        _apply1_body,
        grid=(e_ // be,),
        in_specs=[
            pl.BlockSpec((be, h), lambda i: (i, 0)),
            pl.BlockSpec((be, f), lambda i: (i, 0)),
            pl.BlockSpec((h, f), lambda i: (0, 0)),
        ],
        out_specs=pl.BlockSpec((be, f), lambda i: (i, 0)),
        out_shape=jax.ShapeDtypeStruct((e_, f), F32),
    )(ee, gs, eh)


def _apply2_body(ee_ref, d0_ref, d1_ref, gs_ref, eh_ref, mavg_ref, w_ref):
    alpha = ee_ref[...] / (d0_ref[...] + d1_ref[...] + 1e-9)
    wr = jnp.dot(alpha, eh_ref[...], preferred_element_type=F32) * gs_ref[...]
    w_ref[...] = jnp.dot(wr, mavg_ref[...], preferred_element_type=F32)


def _apply2(ee, d0, d1, gs, eh, mavg, be=4000):
    e_, h = ee.shape
    f = gs.shape[1]
    o = mavg.shape[1]
    return pl.pallas_call(
        _apply2_body,
        grid=(e_ // be,),
        in_specs=[
            pl.BlockSpec((be, h), lambda i: (i, 0)),
            pl.BlockSpec((be, h), lambda i: (i, 0)),
            pl.BlockSpec((be, h), lambda i: (i, 0)),
            pl.BlockSpec((be, f), lambda i: (i, 0)),
            pl.BlockSpec((h, f), lambda i: (0, 0)),
            pl.BlockSpec((f, o), lambda i: (0, 0)),
        ],
        out_specs=pl.BlockSpec((be, o), lambda i: (i, 0)),
        out_shape=jax.ShapeDtypeStruct((e_, o), F32),
    )(ee, d0, d1, gs, eh, mavg)


def _norm_mm_body(o0_ref, o1_ref, d0_ref, d1_ref, eh_ref, w_ref, b_ref, y_ref):
    den = jnp.dot(d0_ref[...] + d1_ref[...], eh_ref[...],
                  preferred_element_type=F32) + 1e-9
    z = (o0_ref[...] + o1_ref[...]) / den + b_ref[...]
    z = jnp.where(z > 0, z, jnp.exp(jnp.minimum(z, 0.0)) - 1.0)
    y_ref[...] = jnp.dot(z, w_ref[...], preferred_element_type=F32)


def _norm_mm(o0, o1, d0, d1, eh, W, b, bn=1024):
    n, f = o0.shape
    h = d0.shape[1]
    f2 = W.shape[1]
    return pl.pallas_call(
        _norm_mm_body,
        grid=(n // bn,),
        in_specs=[
            pl.BlockSpec((bn, f), lambda i: (i, 0)),
            pl.BlockSpec((bn, f), lambda i: (i, 0)),
            pl.BlockSpec((bn, h), lambda i: (i, 0)),
            pl.BlockSpec((bn, h), lambda i: (i, 0)),
            pl.BlockSpec((h, f), lambda i: (0, 0)),
            pl.BlockSpec((f, f2), lambda i: (0, 0)),
            pl.BlockSpec((1, f), lambda i: (0, 0)),
        ],
        out_specs=pl.BlockSpec((bn, f2), lambda i: (i, 0)),
        out_shape=jax.ShapeDtypeStruct((n, f2), F32),
    )(o0, o1, d0, d1, eh, W, b)


def _sc_gather2(table, src, dst, C=CH):
    n, f = table.shape
    e = src.shape[0]
    ew = e // NW
    nchunks = ew // C

    @functools.partial(
        pl.kernel, mesh=_MESH,
        out_type=[jax.ShapeDtypeStruct((e, f), F32)] * 2,
        scratch_types=[
            pltpu.VMEM((C,), jnp.int32),
            pltpu.VMEM((C, f), F32),
            pltpu.SemaphoreType.DMA,
        ],
        compiler_params=_SC_PARAMS,
    )
    def k(table_hbm, src_hbm, dst_hbm, gs_hbm, gd_hbm, idx_v, rows_v, sem):
        wid = lax.axis_index("s") * NC + lax.axis_index("c")

        def body(i, carry):
            base = wid * ew + i * C
            pltpu.sync_copy(src_hbm.at[pl.ds(base, C)], idx_v)
            pltpu.async_copy(table_hbm.at[idx_v], rows_v, sem).wait()
            pltpu.sync_copy(rows_v, gs_hbm.at[pl.ds(base, C)])
            pltpu.sync_copy(dst_hbm.at[pl.ds(base, C)], idx_v)
            pltpu.async_copy(table_hbm.at[idx_v], rows_v, sem).wait()
            pltpu.sync_copy(rows_v, gd_hbm.at[pl.ds(base, C)])
            return carry

        lax.fori_loop(0, nchunks, body, 0)

    return k(table, src, dst)


def _sc_segmax(et, dst, C=CH):
    h, e = et.shape
    ew = e // NW
    nchunks = ew // C
    tw = h * NP
    nr = 4
    rw = tw // nr
    sl = rw // NS

    @functools.partial(
        pl.kernel, mesh=_MESH,
        out_type=jax.ShapeDtypeStruct((NC, tw), F32),
        scratch_types=[
            pltpu.VMEM((tw,), F32),
            pltpu.VMEM((C,), jnp.int32),
            pltpu.VMEM((h, C), F32),
            pltpu.VMEM((sl,), F32),
            pltpu.VMEM((sl,), F32),
            pltpu.VMEM_SHARED((NS, rw), F32),
        ],
        compiler_params=_SC_PARAMS_NL,
    )
    def k(et_hbm, dst_hbm, mp_hbm, loc, dstv, etv, ta, tb, shared):
        c = lax.axis_index("c")
        s = lax.axis_index("s")
        wid = s * NC + c
        neg = jnp.full((16,), -1e30, dtype=F32)

        def init(i, carry):
            loc[pl.ds(i * 16, 16)] = neg
            return carry
        lax.fori_loop(0, tw // 16, init, 0)

        def chunk(i, carry):
            base = wid * ew + i * C
            pltpu.sync_copy(dst_hbm.at[pl.ds(base, C)], dstv)
            pltpu.sync_copy(et_hbm.at[:, pl.ds(base, C)], etv)

            def grp(g, carry2):
                dv = dstv[pl.ds(g * 16, 16)]
                for j in range(h):
                    ev = etv[j, pl.ds(g * 16, 16)]
                    idx = dv + j * NP
                    cur = plsc.load_gather(loc, [idx])
                    plsc.store_scatter(loc, [idx], jnp.maximum(cur, ev))
                return carry2
            lax.fori_loop(0, C // 16, grp, 0)
            return carry
        lax.fori_loop(0, nchunks, chunk, 0)

        for r in range(nr):
            pltpu.sync_copy(loc.at[pl.ds(r * rw, rw)], shared.at[s])
            plsc.subcore_barrier()
            pltpu.sync_copy(shared.at[0, pl.ds(s * sl, sl)], ta)

            def merge(t, carry):
                pltpu.sync_copy(shared.at[t, pl.ds(s * sl, sl)], tb)

                def mv(v, carry2):
                    ta[pl.ds(v * 16, 16)] = jnp.maximum(
                        ta[pl.ds(v * 16, 16)], tb[pl.ds(v * 16, 16)])
                    return carry2
                lax.fori_loop(0, sl // 16, mv, 0)
                return carry
            lax.fori_loop(1, NS, merge, 0)
            pltpu.sync_copy(ta, mp_hbm.at[c, pl.ds(r * rw + s * sl, sl)])
            plsc.subcore_barrier()

    return k(et, dst)


def _sc_eepass(et, dst, mpart, C=CH):
    h, e = et.shape
    ew = e // NW
    nchunks = ew // C
    tw = h * NP
    sl = tw // NS

    @functools.partial(
        pl.kernel, mesh=_MESH,
        out_type=jax.ShapeDtypeStruct((e, h), F32),
        scratch_types=[
            pltpu.VMEM((tw,), F32),
            pltpu.VMEM((sl,), F32),
            pltpu.VMEM((sl,), F32),
            pltpu.VMEM((C,), jnp.int32),
            pltpu.VMEM((h, C), F32),
            pltpu.VMEM((C, h), F32),
        ],
        compiler_params=_SC_PARAMS_NL,
    )
    def k(et_hbm, dst_hbm, mp_hbm, ee_hbm, mloc, tb0, tb1, dstv, etv, eev):
        c = lax.axis_index("c")
        s = lax.axis_index("s")
        wid = s * NC + c
        lanes = lax.iota(jnp.int32, 16)

        def stage(t, carry):
            pltpu.sync_copy(mp_hbm.at[0, pl.ds(t * sl, sl)], tb0)
            pltpu.sync_copy(mp_hbm.at[1, pl.ds(t * sl, sl)], tb1)

            def mv(v, carry2):
                mloc[pl.ds(t * sl + v * 16, 16)] = jnp.maximum(
                    tb0[pl.ds(v * 16, 16)], tb1[pl.ds(v * 16, 16)])
                return carry2
            lax.fori_loop(0, sl // 16, mv, 0)
            return carry
        lax.fori_loop(0, NS, stage, 0)

        def chunk(i, carry):
            base = wid * ew + i * C
            pltpu.sync_copy(dst_hbm.at[pl.ds(base, C)], dstv)
            pltpu.sync_copy(et_hbm.at[:, pl.ds(base, C)], etv)

            def grp(g, carry2):
                dv = dstv[pl.ds(g * 16, 16)]
                rows = g * 16 + lanes
                for j in range(h):
                    idx = dv + j * NP
                    mv2 = plsc.load_gather(mloc, [idx])
                    ee = jnp.exp(etv[j, pl.ds(g * 16, 16)] - mv2)
                    cols = jnp.full((16,), j, dtype=jnp.int32)
                    plsc.store_scatter(eev, [rows, cols], ee)
                return carry2
            lax.fori_loop(0, C // 16, grp, 0)
            pltpu.sync_copy(eev, ee_hbm.at[pl.ds(base, C)])
            return carry
        lax.fori_loop(0, nchunks, chunk, 0)

    return k(et, dst, mpart)


def _sc_scatter2(wrows, ee, dst, zf, z8, C=CH):
    e, f = wrows.shape
    h = ee.shape[1]
    ew = e // NW
    nchunks = ew // C
    sl = NP // NS

    @functools.partial(
        pl.kernel, mesh=_MESH,
        out_type=[jax.ShapeDtypeStruct((NC, NP, f), F32),
                  jax.ShapeDtypeStruct((NC, NP, h), F32)],
        scratch_types=[
            pltpu.VMEM((C,), jnp.int32),
            pltpu.VMEM((C, f), F32),
            pltpu.VMEM((C, h), F32),
            pltpu.VMEM_SHARED((NP, f), F32),
            pltpu.VMEM_SHARED((NP, h), F32),
        ],
        compiler_params=_SC_PARAMS,
    )
    def k(wr_hbm, ee_hbm, dst_hbm, zf_hbm, z8_hbm, op_hbm, dp_hbm,
          idx_v, wv, ev, acc_o, acc_d):
        c = lax.axis_index("c")
        s = lax.axis_index("s")
        wid = s * NC + c
        pltpu.sync_copy(zf_hbm.at[pl.ds(s * sl, sl)], acc_o.at[pl.ds(s * sl, sl)])
        pltpu.sync_copy(z8_hbm.at[pl.ds(s * sl, sl)], acc_d.at[pl.ds(s * sl, sl)])
        plsc.subcore_barrier()

        def body(i, carry):
            base = wid * ew + i * C
            pltpu.sync_copy(dst_hbm.at[pl.ds(base, C)], idx_v)
            pltpu.sync_copy(wr_hbm.at[pl.ds(base, C)], wv)
            pltpu.sync_copy(ee_hbm.at[pl.ds(base, C)], ev)
            pltpu.sync_copy(wv, acc_o.at[idx_v], add=True)
            pltpu.sync_copy(ev, acc_d.at[idx_v], add=True)
            return carry

        lax.fori_loop(0, nchunks, body, 0)
        plsc.subcore_barrier()
        pltpu.sync_copy(acc_o.at[pl.ds(s * sl, sl)], op_hbm.at[c, pl.ds(s * sl, sl)])
        pltpu.sync_copy(acc_d.at[pl.ds(s * sl, sl)], dp_hbm.at[c, pl.ds(s * sl, sl)])

    return k(wrows, ee, dst, zf, z8)


def _sc_scatter1(rows, dst, zrows, C=CH):
    e, f = rows.shape
    ew = e // NW
    nchunks = ew // C
    sl = NP // NS

    @functools.partial(
        pl.kernel, mesh=_MESH,
        out_type=jax.ShapeDtypeStruct((NC, NP, f), F32),
        scratch_types=[
            pltpu.VMEM((C,), jnp.int32),
            pltpu.VMEM((C, f), F32),
            pltpu.VMEM_SHARED((NP, f), F32),
        ],
        compiler_params=_SC_PARAMS,
    )
    def k(r_hbm, dst_hbm, z_hbm, op_hbm, idx_v, rv, acc):
        c = lax.axis_index("c")
        s = lax.axis_index("s")
        wid = s * NC + c
        pltpu.sync_copy(z_hbm.at[pl.ds(s * sl, sl)], acc.at[pl.ds(s * sl, sl)])
        plsc.subcore_barrier()

        def body(i, carry):
            base = wid * ew + i * C
            pltpu.sync_copy(dst_hbm.at[pl.ds(base, C)], idx_v)
            pltpu.sync_copy(r_hbm.at[pl.ds(base, C)], rv)
            pltpu.sync_copy(rv, acc.at[idx_v], add=True)
            return carry

        lax.fori_loop(0, nchunks, body, 0)
        plsc.subcore_barrier()
        pltpu.sync_copy(acc.at[pl.ds(s * sl, sl)], op_hbm.at[c, pl.ds(s * sl, sl)])

    return k(rows, dst, zrows)


def _sc_gather_d2(dpart, dst, C=CH):
    _, n, h = dpart.shape
    e = dst.shape[0]
    ew = e // NW
    nchunks = ew // C

    @functools.partial(
        pl.kernel, mesh=_MESH,
        out_type=[jax.ShapeDtypeStruct((e, h), F32)] * 2,
        scratch_types=[
            pltpu.VMEM((C,), jnp.int32),
            pltpu.VMEM((C, h), F32),
            pltpu.SemaphoreType.DMA,
        ],
        compiler_params=_SC_PARAMS,
    )
    def k(dp_hbm, dst_hbm, g0_hbm, g1_hbm, idx_v, rv, sem):
        wid = lax.axis_index("s") * NC + lax.axis_index("c")

        def body(i, carry):
            base = wid * ew + i * C
            pltpu.sync_copy(dst_hbm.at[pl.ds(base, C)], idx_v)
            pltpu.async_copy(dp_hbm.at[0].at[idx_v], rv, sem).wait()
            pltpu.sync_copy(rv, g0_hbm.at[pl.ds(base, C)])
            pltpu.async_copy(dp_hbm.at[1].at[idx_v], rv, sem).wait()
            pltpu.sync_copy(rv, g1_hbm.at[pl.ds(base, C)])
            return carry

        lax.fori_loop(0, nchunks, body, 0)

    return k(dpart, dst)


def _sc_take_out(opart, b2, idxpad):
    _, n, f = opart.shape
    q = idxpad.shape[0]
    qw = q // NW

    @functools.partial(
        pl.kernel, mesh=_MESH,
        out_type=jax.ShapeDtypeStruct((q, f), F32),
        scratch_types=[
            pltpu.VMEM((qw,), jnp.int32),
            pltpu.VMEM((qw, f), F32),
            pltpu.VMEM((qw, f), F32),
            pltpu.VMEM((f,), F32),
            pltpu.SemaphoreType.DMA,
        ],
        compiler_params=_SC_PARAMS,
    )
    def k(op_hbm, b2_hbm, idx_hbm, out_hbm, idx_v, r0, r1, bv, sem):
        wid = lax.axis_index("s") * NC + lax.axis_index("c")
        base = wid * qw
        pltpu.sync_copy(b2_hbm, bv)
        pltpu.sync_copy(idx_hbm.at[pl.ds(base, qw)], idx_v)
        pltpu.async_copy(op_hbm.at[0].at[idx_v], r0, sem).wait()
        pltpu.async_copy(op_hbm.at[1].at[idx_v], r1, sem).wait()
        b = bv[...]

        def rowadd(r, carry):
            r0[r, :] = r0[r, :] + r1[r, :] + b
            return carry
        lax.fori_loop(0, qw, rowadd, 0)
        pltpu.sync_copy(r0, out_hbm.at[pl.ds(base, qw)])

    return k(opart, b2, idxpad)


def kernel(x, edge_index, indices, W1, a1, b1, W2, a2, b2):
    n = x.shape[0]
    h1c, u1 = a1.shape
    h2c, u2 = a2.shape
    f1 = h1c * u1
    f2 = h2c * u2
    src = edge_index[0]
    dst = edge_index[1]

    eye1 = jnp.eye(h1c, dtype=F32)
    ab1 = (eye1[:, None, :] * a1[:, :, None]).reshape(f1, h1c)
    eh1 = jnp.repeat(eye1, u1, axis=1)
    eye2 = jnp.eye(h2c, dtype=F32)
    ab2 = (eye2[:, None, :] * a2[:, :, None]).reshape(f2, h2c)
    eh2 = jnp.repeat(eye2, u2, axis=1)
    mavg = jnp.tile(jnp.eye(u2, dtype=F32), (h2c, 1)) / h2c

    zf1 = jnp.zeros((NP, f1), F32)
    z8 = jnp.zeros((NP, h1c), F32)
    zo = jnp.zeros((NP, u2), F32)

    h1 = _matmul(x, W1)
    gs1, gd1 = _sc_gather2(h1, src, dst)
    et1 = _edge_eT(gs1, gd1, ab1)
    mp1 = _sc_segmax(et1, dst)
    ee1 = _sc_eepass(et1, dst, mp1)
    w1 = _apply1(ee1, gs1, eh1)
    op1, dp1 = _sc_scatter2(w1, ee1, dst, zf1, z8)
    h2 = _norm_mm(op1[0], op1[1], dp1[0], dp1[1], eh1, W2,
                  b1.reshape(1, f1))

    gs2, gd2 = _sc_gather2(h2, src, dst)
    et2 = _edge_eT(gs2, gd2, ab2)
    mp2 = _sc_segmax(et2, dst)
    ee2 = _sc_eepass(et2, dst, mp2)
    dp2 = _sc_scatter1(ee2, dst, z8)
    d0g, d1g = _sc_gather_d2(dp2, dst)
    wred = _apply2(ee2, d0g, d1g, gs2, eh2, mavg)
    opr = _sc_scatter1(wred, dst, zo)

    q = indices.shape[0]
    qpad = NW * ((q + NW - 1) // NW)
    idxpad = jnp.concatenate(
        [indices, jnp.zeros((qpad - q,), jnp.int32)])
    out = _sc_take_out(opr, b2, idxpad)
    return out[:q]


# speedup vs baseline: 30.2711x; 1.0168x over previous
import functools

import jax
import jax.numpy as jnp
from jax import lax
from jax.experimental import pallas as pl
from jax.experimental.pallas import tpu as pltpu, tpu_sc as plsc

F32 = jnp.float32
NC, NS = 2, 16
NW = NC * NS
NP = 10240
CH = 400

_SC_PARAMS = pltpu.CompilerParams(use_tc_tiling_on_sc=False)
_SC_PARAMS_NL = pltpu.CompilerParams(
    use_tc_tiling_on_sc=False, needs_layout_passes=False)
_MESH = plsc.VectorSubcoreMesh(core_axis_name="c", subcore_axis_name="s")


def _mm_body(x_ref, w_ref, o_ref):
    o_ref[...] = jnp.dot(x_ref[...], w_ref[...], preferred_element_type=F32)


def _matmul(x, W, bn=1000):
    n, k = x.shape
    f = W.shape[1]
    return pl.pallas_call(
        _mm_body,
        grid=(n // bn,),
        in_specs=[
            pl.BlockSpec((bn, k), lambda i: (i, 0)),
            pl.BlockSpec((k, f), lambda i: (0, 0)),
        ],
        out_specs=pl.BlockSpec((bn, f), lambda i: (i, 0)),
        out_shape=jax.ShapeDtypeStruct((n, f), F32),
    )(x, W)


def _edge_e_body(gs_ref, gd_ref, ab_ref, e_ref):
    m = gs_ref[...] + gd_ref[...]
    m = jnp.where(m >= 0, m, 0.2 * m)
    e_ref[...] = lax.dot_general(ab_ref[...], m, (((0,), (1,)), ((), ())),
                                 preferred_element_type=F32)


def _edge_eT(gs, gd, ablock, be=6400):
    e_, f = gs.shape
    h = ablock.shape[1]
    return pl.pallas_call(
        _edge_e_body,
        grid=(e_ // be,),
        in_specs=[
            pl.BlockSpec((be, f), lambda i: (i, 0)),
            pl.BlockSpec((be, f), lambda i: (i, 0)),
            pl.BlockSpec((f, h), lambda i: (0, 0)),
        ],
        out_specs=pl.BlockSpec((h, be), lambda i: (0, i)),
        out_shape=jax.ShapeDtypeStruct((h, e_), F32),
    )(gs, gd, ablock)


def _apply1_body(ee_ref, gs_ref, eh_ref, w_ref):
    w_ref[...] = jnp.dot(ee_ref[...], eh_ref[...],
                         preferred_element_type=F32) * gs_ref[...]


def _apply1(ee, gs, eh, be=4000):
    e_, h = ee.shape
    f = gs.shape[1]
    return pl.pallas_call(
        _apply1_body,
        grid=(e_ // be,),
        in_specs=[
            pl.BlockSpec((be, h), lambda i: (i, 0)),
            pl.BlockSpec((be, f), lambda i: (i, 0)),
            pl.BlockSpec((h, f), lambda i: (0, 0)),
        ],
        out_specs=pl.BlockSpec((be, f), lambda i: (i, 0)),
        out_shape=jax.ShapeDtypeStruct((e_, f), F32),
    )(ee, gs, eh)


def _apply2_body(ee_ref, d0_ref, d1_ref, gs_ref, eh_ref, mavg_ref, w_ref):
    alpha = ee_ref[...] / (d0_ref[...] + d1_ref[...] + 1e-9)
    wr = jnp.dot(alpha, eh_ref[...], preferred_element_type=F32) * gs_ref[...]
    w_ref[...] = jnp.dot(wr, mavg_ref[...], preferred_element_type=F32)


def _apply2(ee, d0, d1, gs, eh, mavg, be=4000):
    e_, h = ee.shape
    f = gs.shape[1]
    o = mavg.shape[1]
    return pl.pallas_call(
        _apply2_body,
        grid=(e_ // be,),
        in_specs=[
            pl.BlockSpec((be, h), lambda i: (i, 0)),
            pl.BlockSpec((be, h), lambda i: (i, 0)),
            pl.BlockSpec((be, h), lambda i: (i, 0)),
            pl.BlockSpec((be, f), lambda i: (i, 0)),
            pl.BlockSpec((h, f), lambda i: (0, 0)),
            pl.BlockSpec((f, o), lambda i: (0, 0)),
        ],
        out_specs=pl.BlockSpec((be, o), lambda i: (i, 0)),
        out_shape=jax.ShapeDtypeStruct((e_, o), F32),
    )(ee, d0, d1, gs, eh, mavg)


def _norm_mm_body(o0_ref, o1_ref, d0_ref, d1_ref, eh_ref, w_ref, b_ref, y_ref):
    den = jnp.dot(d0_ref[...] + d1_ref[...], eh_ref[...],
                  preferred_element_type=F32) + 1e-9
    z = (o0_ref[...] + o1_ref[...]) / den + b_ref[...]
    z = jnp.where(z > 0, z, jnp.exp(jnp.minimum(z, 0.0)) - 1.0)
    y_ref[...] = jnp.dot(z, w_ref[...], preferred_element_type=F32)


def _norm_mm(o0, o1, d0, d1, eh, W, b, bn=1024):
    n, f = o0.shape
    h = d0.shape[1]
    f2 = W.shape[1]
    return pl.pallas_call(
        _norm_mm_body,
        grid=(n // bn,),
        in_specs=[
            pl.BlockSpec((bn, f), lambda i: (i, 0)),
            pl.BlockSpec((bn, f), lambda i: (i, 0)),
            pl.BlockSpec((bn, h), lambda i: (i, 0)),
            pl.BlockSpec((bn, h), lambda i: (i, 0)),
            pl.BlockSpec((h, f), lambda i: (0, 0)),
            pl.BlockSpec((f, f2), lambda i: (0, 0)),
            pl.BlockSpec((1, f), lambda i: (0, 0)),
        ],
        out_specs=pl.BlockSpec((bn, f2), lambda i: (i, 0)),
        out_shape=jax.ShapeDtypeStruct((n, f2), F32),
    )(o0, o1, d0, d1, eh, W, b)


def _sc_gather2(table, src, dst, C=400):
    n, f = table.shape
    e = src.shape[0]
    ew = e // NW
    nchunks = ew // C

    @functools.partial(
        pl.kernel, mesh=_MESH,
        out_type=[jax.ShapeDtypeStruct((e, f), F32)] * 2,
        scratch_types=[
            pltpu.VMEM((C,), jnp.int32),
            pltpu.VMEM((C, f), F32),
            pltpu.SemaphoreType.DMA,
        ],
        compiler_params=_SC_PARAMS,
    )
    def k(table_hbm, src_hbm, dst_hbm, gs_hbm, gd_hbm, idx_v, rows_v, sem):
        wid = lax.axis_index("s") * NC + lax.axis_index("c")

        def body(i, carry):
            base = wid * ew + i * C
            pltpu.sync_copy(src_hbm.at[pl.ds(base, C)], idx_v)
            pltpu.async_copy(table_hbm.at[idx_v], rows_v, sem).wait()
            pltpu.sync_copy(rows_v, gs_hbm.at[pl.ds(base, C)])
            pltpu.sync_copy(dst_hbm.at[pl.ds(base, C)], idx_v)
            pltpu.async_copy(table_hbm.at[idx_v], rows_v, sem).wait()
            pltpu.sync_copy(rows_v, gd_hbm.at[pl.ds(base, C)])
            return carry

        lax.fori_loop(0, nchunks, body, 0)

    return k(table, src, dst)


def _sc_segmax(et, dst, C=400):
    h, e = et.shape
    ew = e // NW
    nchunks = ew // C
    tw = h * NP
    nr = 8
    rw = tw // nr
    sl = rw // NS

    @functools.partial(
        pl.kernel, mesh=_MESH,
        out_type=jax.ShapeDtypeStruct((NC, tw), F32),
        scratch_types=[
            pltpu.VMEM((tw,), F32),
            pltpu.VMEM((C,), jnp.int32),
            pltpu.VMEM((h, C), F32),
            pltpu.VMEM((sl,), F32),
            pltpu.VMEM((sl,), F32),
            pltpu.VMEM_SHARED((NS, rw), F32),
        ],
        compiler_params=_SC_PARAMS_NL,
    )
    def k(et_hbm, dst_hbm, mp_hbm, loc, dstv, etv, ta, tb, shared):
        c = lax.axis_index("c")
        s = lax.axis_index("s")
        wid = s * NC + c
        neg = jnp.full((16,), -1e30, dtype=F32)

        def init(i, carry):
            loc[pl.ds(i * 16, 16)] = neg
            return carry
        lax.fori_loop(0, tw // 16, init, 0)

        def chunk(i, carry):
            base = wid * ew + i * C
            pltpu.sync_copy(dst_hbm.at[pl.ds(base, C)], dstv)
            pltpu.sync_copy(et_hbm.at[:, pl.ds(base, C)], etv)

            def grp(g, carry2):
                dv = dstv[pl.ds(g * 16, 16)]
                for j in range(h):
                    ev = etv[j, pl.ds(g * 16, 16)]
                    idx = dv + j * NP
                    cur = plsc.load_gather(loc, [idx])
                    plsc.store_scatter(loc, [idx], jnp.maximum(cur, ev))
                return carry2
            lax.fori_loop(0, C // 16, grp, 0)
            return carry
        lax.fori_loop(0, nchunks, chunk, 0)

        for r in range(nr):
            pltpu.sync_copy(loc.at[pl.ds(r * rw, rw)], shared.at[s])
            plsc.subcore_barrier()
            pltpu.sync_copy(shared.at[0, pl.ds(s * sl, sl)], ta)

            def merge(t, carry):
                pltpu.sync_copy(shared.at[t, pl.ds(s * sl, sl)], tb)

                def mv(v, carry2):
                    ta[pl.ds(v * 16, 16)] = jnp.maximum(
                        ta[pl.ds(v * 16, 16)], tb[pl.ds(v * 16, 16)])
                    return carry2
                lax.fori_loop(0, sl // 16, mv, 0)
                return carry
            lax.fori_loop(1, NS, merge, 0)
            pltpu.sync_copy(ta, mp_hbm.at[c, pl.ds(r * rw + s * sl, sl)])
            plsc.subcore_barrier()

    return k(et, dst)


def _sc_eepass(et, dst, mpart, z8, C=400):
    h, e = et.shape
    ew = e // NW
    nchunks = ew // C
    tw = h * NP
    sl = tw // NS
    sn = NP // NS

    @functools.partial(
        pl.kernel, mesh=_MESH,
        out_type=[jax.ShapeDtypeStruct((e, h), F32),
                  jax.ShapeDtypeStruct((NC, NP, h), F32)],
        scratch_types=[
            pltpu.VMEM((tw,), F32),
            pltpu.VMEM((sl,), F32),
            pltpu.VMEM((sl,), F32),
            pltpu.VMEM((C,), jnp.int32),
            pltpu.VMEM((h, C), F32),
            pltpu.VMEM((C, h), F32),
            pltpu.VMEM_SHARED((NP, h), F32),
        ],
        compiler_params=_SC_PARAMS_NL,
    )
    def k(et_hbm, dst_hbm, mp_hbm, z8_hbm, ee_hbm, dp_hbm,
          mloc, tb0, tb1, dstv, etv, eev, acc_d):
        c = lax.axis_index("c")
        s = lax.axis_index("s")
        wid = s * NC + c
        lanes = lax.iota(jnp.int32, 16)
        pltpu.sync_copy(z8_hbm.at[pl.ds(s * sn, sn)], acc_d.at[pl.ds(s * sn, sn)])

        def stage(t, carry):
            pltpu.sync_copy(mp_hbm.at[0, pl.ds(t * sl, sl)], tb0)
            pltpu.sync_copy(mp_hbm.at[1, pl.ds(t * sl, sl)], tb1)

            def mv(v, carry2):
                mloc[pl.ds(t * sl + v * 16, 16)] = jnp.maximum(
                    tb0[pl.ds(v * 16, 16)], tb1[pl.ds(v * 16, 16)])
                return carry2
            lax.fori_loop(0, sl // 16, mv, 0)
            return carry
        lax.fori_loop(0, NS, stage, 0)
        plsc.subcore_barrier()

        def chunk(i, carry):
            base = wid * ew + i * C
            pltpu.sync_copy(dst_hbm.at[pl.ds(base, C)], dstv)
            pltpu.sync_copy(et_hbm.at[:, pl.ds(base, C)], etv)

            def grp(g, carry2):
                dv = dstv[pl.ds(g * 16, 16)]
                rows = g * 16 + lanes
                for j in range(h):
                    idx = dv + j * NP
                    mv2 = plsc.load_gather(mloc, [idx])
                    ee = jnp.exp(etv[j, pl.ds(g * 16, 16)] - mv2)
                    cols = jnp.full((16,), j, dtype=jnp.int32)
                    plsc.store_scatter(eev, [rows, cols], ee)
                return carry2
            lax.fori_loop(0, C // 16, grp, 0)
            pltpu.sync_copy(eev, ee_hbm.at[pl.ds(base, C)])
            pltpu.sync_copy(eev, acc_d.at[dstv], add=True)
            return carry
        lax.fori_loop(0, nchunks, chunk, 0)

        plsc.subcore_barrier()
        pltpu.sync_copy(acc_d.at[pl.ds(s * sn, sn)], dp_hbm.at[c, pl.ds(s * sn, sn)])

    return k(et, dst, mpart, z8)


def _sc_scatter1(rows, dst, zrows, C=400):
    e, f = rows.shape
    ew = e // NW
    nchunks = ew // C
    sl = NP // NS

    @functools.partial(
        pl.kernel, mesh=_MESH,
        out_type=jax.ShapeDtypeStruct((NC, NP, f), F32),
        scratch_types=[
            pltpu.VMEM((C,), jnp.int32),
            pltpu.VMEM((C, f), F32),
            pltpu.VMEM_SHARED((NP, f), F32),
        ],
        compiler_params=_SC_PARAMS,
    )
    def k(r_hbm, dst_hbm, z_hbm, op_hbm, idx_v, rv, acc):
        c = lax.axis_index("c")
        s = lax.axis_index("s")
        wid = s * NC + c
        pltpu.sync_copy(z_hbm.at[pl.ds(s * sl, sl)], acc.at[pl.ds(s * sl, sl)])
        plsc.subcore_barrier()

        def body(i, carry):
            base = wid * ew + i * C
            pltpu.sync_copy(dst_hbm.at[pl.ds(base, C)], idx_v)
            pltpu.sync_copy(r_hbm.at[pl.ds(base, C)], rv)
            pltpu.sync_copy(rv, acc.at[idx_v], add=True)
            return carry

        lax.fori_loop(0, nchunks, body, 0)
        plsc.subcore_barrier()
        pltpu.sync_copy(acc.at[pl.ds(s * sl, sl)], op_hbm.at[c, pl.ds(s * sl, sl)])

    return k(rows, dst, zrows)


def _sc_gather_d2(dpart, dst, C=400):
    _, n, h = dpart.shape
    e = dst.shape[0]
    ew = e // NW
    nchunks = ew // C

    @functools.partial(
        pl.kernel, mesh=_MESH,
        out_type=[jax.ShapeDtypeStruct((e, h), F32)] * 2,
        scratch_types=[
            pltpu.VMEM((C,), jnp.int32),
            pltpu.VMEM((C, h), F32),
            pltpu.SemaphoreType.DMA,
        ],
        compiler_params=_SC_PARAMS,
    )
    def k(dp_hbm, dst_hbm, g0_hbm, g1_hbm, idx_v, rv, sem):
        wid = lax.axis_index("s") * NC + lax.axis_index("c")

        def body(i, carry):
            base = wid * ew + i * C
            pltpu.sync_copy(dst_hbm.at[pl.ds(base, C)], idx_v)
            pltpu.async_copy(dp_hbm.at[0].at[idx_v], rv, sem).wait()
            pltpu.sync_copy(rv, g0_hbm.at[pl.ds(base, C)])
            pltpu.async_copy(dp_hbm.at[1].at[idx_v], rv, sem).wait()
            pltpu.sync_copy(rv, g1_hbm.at[pl.ds(base, C)])
            return carry

        lax.fori_loop(0, nchunks, body, 0)

    return k(dpart, dst)


def _sc_take_out(opart, b2, idxpad):
    _, n, f = opart.shape
    q = idxpad.shape[0]
    qw = q // NW

    @functools.partial(
        pl.kernel, mesh=_MESH,
        out_type=jax.ShapeDtypeStruct((q, f), F32),
        scratch_types=[
            pltpu.VMEM((qw,), jnp.int32),
            pltpu.VMEM((qw, f), F32),
            pltpu.VMEM((qw, f), F32),
            pltpu.VMEM((f,), F32),
            pltpu.SemaphoreType.DMA,
        ],
        compiler_params=_SC_PARAMS,
    )
    def k(op_hbm, b2_hbm, idx_hbm, out_hbm, idx_v, r0, r1, bv, sem):
        wid = lax.axis_index("s") * NC + lax.axis_index("c")
        base = wid * qw
        pltpu.sync_copy(b2_hbm, bv)
        pltpu.sync_copy(idx_hbm.at[pl.ds(base, qw)], idx_v)
        pltpu.async_copy(op_hbm.at[0].at[idx_v], r0, sem).wait()
        pltpu.async_copy(op_hbm.at[1].at[idx_v], r1, sem).wait()
        b = bv[...]

        def rowadd(r, carry):
            r0[r, :] = r0[r, :] + r1[r, :] + b
            return carry
        lax.fori_loop(0, qw, rowadd, 0)
        pltpu.sync_copy(r0, out_hbm.at[pl.ds(base, qw)])

    return k(opart, b2, idxpad)


def kernel(x, edge_index, indices, W1, a1, b1, W2, a2, b2):
    n = x.shape[0]
    h1c, u1 = a1.shape
    h2c, u2 = a2.shape
    f1 = h1c * u1
    f2 = h2c * u2
    src = edge_index[0]
    dst = edge_index[1]

    eye1 = jnp.eye(h1c, dtype=F32)
    ab1 = (eye1[:, None, :] * a1[:, :, None]).reshape(f1, h1c)
    eh1 = jnp.repeat(eye1, u1, axis=1)
    eye2 = jnp.eye(h2c, dtype=F32)
    ab2 = (eye2[:, None, :] * a2[:, :, None]).reshape(f2, h2c)
    eh2 = jnp.repeat(eye2, u2, axis=1)
    mavg = jnp.tile(jnp.eye(u2, dtype=F32), (h2c, 1)) / h2c

    zf1 = jnp.zeros((NP, f1), F32)
    z8 = jnp.zeros((NP, h1c), F32)
    zo = jnp.zeros((NP, u2), F32)

    h1 = _matmul(x, W1)
    gs1, gd1 = _sc_gather2(h1, src, dst)
    et1 = _edge_eT(gs1, gd1, ab1)
    mp1 = _sc_segmax(et1, dst)
    ee1, dp1 = _sc_eepass(et1, dst, mp1, z8)
    w1 = _apply1(ee1, gs1, eh1)
    op1 = _sc_scatter1(w1, dst, zf1)
    h2 = _norm_mm(op1[0], op1[1], dp1[0], dp1[1], eh1, W2,
                  b1.reshape(1, f1))

    gs2, gd2 = _sc_gather2(h2, src, dst)
    et2 = _edge_eT(gs2, gd2, ab2)
    mp2 = _sc_segmax(et2, dst)
    ee2, dp2 = _sc_eepass(et2, dst, mp2, z8)
    d0g, d1g = _sc_gather_d2(dp2, dst)
    wred = _apply2(ee2, d0g, d1g, gs2, eh2, mavg)
    opr = _sc_scatter1(wred, dst, zo, C=400)

    q = indices.shape[0]
    qpad = NW * ((q + NW - 1) // NW)
    idxpad = jnp.concatenate(
        [indices, jnp.zeros((qpad - q,), jnp.int32)])
    out = _sc_take_out(opr, b2, idxpad)
    return out[:q]


# speedup vs baseline: 32.5266x; 1.0745x over previous
import functools

import jax
import jax.numpy as jnp
from jax import lax
from jax.experimental import pallas as pl
from jax.experimental.pallas import tpu as pltpu, tpu_sc as plsc

F32 = jnp.float32
NC, NS = 2, 16
NW = NC * NS
NP = 10240
CH = 400

_SC_PARAMS = pltpu.CompilerParams(use_tc_tiling_on_sc=False)
_SC_PARAMS_NL = pltpu.CompilerParams(
    use_tc_tiling_on_sc=False, needs_layout_passes=False)
_MESH = plsc.VectorSubcoreMesh(core_axis_name="c", subcore_axis_name="s")


def _mm_body(x_ref, w_ref, o_ref):
    o_ref[...] = jnp.dot(x_ref[...], w_ref[...], preferred_element_type=F32)


def _matmul(x, W, bn=1000):
    n, k = x.shape
    f = W.shape[1]
    return pl.pallas_call(
        _mm_body,
        grid=(n // bn,),
        in_specs=[
            pl.BlockSpec((bn, k), lambda i: (i, 0)),
            pl.BlockSpec((k, f), lambda i: (0, 0)),
        ],
        out_specs=pl.BlockSpec((bn, f), lambda i: (i, 0)),
        out_shape=jax.ShapeDtypeStruct((n, f), F32),
    )(x, W)


def _edge_e_body(gs_ref, gd_ref, ab_ref, e_ref):
    m = gs_ref[...] + gd_ref[...]
    m = jnp.where(m >= 0, m, 0.2 * m)
    e_ref[...] = lax.dot_general(ab_ref[...], m, (((0,), (1,)), ((), ())),
                                 preferred_element_type=F32)


def _edge_eT(gs, gd, ablock, be=6400):
    e_, f = gs.shape
    h = ablock.shape[1]
    return pl.pallas_call(
        _edge_e_body,
        grid=(e_ // be,),
        in_specs=[
            pl.BlockSpec((be, f), lambda i: (i, 0)),
            pl.BlockSpec((be, f), lambda i: (i, 0)),
            pl.BlockSpec((f, h), lambda i: (0, 0)),
        ],
        out_specs=pl.BlockSpec((h, be), lambda i: (0, i)),
        out_shape=jax.ShapeDtypeStruct((h, e_), F32),
    )(gs, gd, ablock)


def _apply1_body(ee_ref, gs_ref, eh_ref, w_ref):
    w_ref[...] = jnp.dot(ee_ref[...], eh_ref[...],
                         preferred_element_type=F32) * gs_ref[...]


def _apply1(ee, gs, eh, be=4000):
    e_, h = ee.shape
    f = gs.shape[1]
    return pl.pallas_call(
        _apply1_body,
        grid=(e_ // be,),
        in_specs=[
            pl.BlockSpec((be, h), lambda i: (i, 0)),
            pl.BlockSpec((be, f), lambda i: (i, 0)),
            pl.BlockSpec((h, f), lambda i: (0, 0)),
        ],
        out_specs=pl.BlockSpec((be, f), lambda i: (i, 0)),
        out_shape=jax.ShapeDtypeStruct((e_, f), F32),
    )(ee, gs, eh)


def _apply2_body(ee_ref, d0_ref, d1_ref, gs_ref, eh_ref, mavg_ref, w_ref):
    alpha = ee_ref[...] / (d0_ref[...] + d1_ref[...] + 1e-9)
    wr = jnp.dot(alpha, eh_ref[...], preferred_element_type=F32) * gs_ref[...]
    w_ref[...] = jnp.dot(wr, mavg_ref[...], preferred_element_type=F32)


def _apply2(ee, d0, d1, gs, eh, mavg, be=4000):
    e_, h = ee.shape
    f = gs.shape[1]
    o = mavg.shape[1]
    return pl.pallas_call(
        _apply2_body,
        grid=(e_ // be,),
        in_specs=[
            pl.BlockSpec((be, h), lambda i: (i, 0)),
            pl.BlockSpec((be, h), lambda i: (i, 0)),
            pl.BlockSpec((be, h), lambda i: (i, 0)),
            pl.BlockSpec((be, f), lambda i: (i, 0)),
            pl.BlockSpec((h, f), lambda i: (0, 0)),
            pl.BlockSpec((f, o), lambda i: (0, 0)),
        ],
        out_specs=pl.BlockSpec((be, o), lambda i: (i, 0)),
        out_shape=jax.ShapeDtypeStruct((e_, o), F32),
    )(ee, d0, d1, gs, eh, mavg)


def _norm_mm_body(o0_ref, o1_ref, d0_ref, d1_ref, eh_ref, w_ref, b_ref, y_ref):
    den = jnp.dot(d0_ref[...] + d1_ref[...], eh_ref[...],
                  preferred_element_type=F32) + 1e-9
    z = (o0_ref[...] + o1_ref[...]) / den + b_ref[...]
    z = jnp.where(z > 0, z, jnp.exp(jnp.minimum(z, 0.0)) - 1.0)
    y_ref[...] = jnp.dot(z, w_ref[...], preferred_element_type=F32)


def _norm_mm(o0, o1, d0, d1, eh, W, b, bn=1024):
    n, f = o0.shape
    h = d0.shape[1]
    f2 = W.shape[1]
    return pl.pallas_call(
        _norm_mm_body,
        grid=(n // bn,),
        in_specs=[
            pl.BlockSpec((bn, f), lambda i: (i, 0)),
            pl.BlockSpec((bn, f), lambda i: (i, 0)),
            pl.BlockSpec((bn, h), lambda i: (i, 0)),
            pl.BlockSpec((bn, h), lambda i: (i, 0)),
            pl.BlockSpec((h, f), lambda i: (0, 0)),
            pl.BlockSpec((f, f2), lambda i: (0, 0)),
            pl.BlockSpec((1, f), lambda i: (0, 0)),
        ],
        out_specs=pl.BlockSpec((bn, f2), lambda i: (i, 0)),
        out_shape=jax.ShapeDtypeStruct((n, f2), F32),
    )(o0, o1, d0, d1, eh, W, b)


def _sc_gather2(table, src, dst, C=1000):
    n, f = table.shape
    e = src.shape[0]
    ew = e // NW
    nchunks = ew // C

    @functools.partial(
        pl.kernel, mesh=_MESH,
        out_type=[jax.ShapeDtypeStruct((e, f), F32)] * 2,
        scratch_types=[
            pltpu.VMEM((C,), jnp.int32),
            pltpu.VMEM((C, f), F32),
            pltpu.SemaphoreType.DMA,
        ],
        compiler_params=_SC_PARAMS,
    )
    def k(table_hbm, src_hbm, dst_hbm, gs_hbm, gd_hbm, idx_v, rows_v, sem):
        wid = lax.axis_index("s") * NC + lax.axis_index("c")

        def body(i, carry):
            base = wid * ew + i * C
            pltpu.sync_copy(src_hbm.at[pl.ds(base, C)], idx_v)
            pltpu.async_copy(table_hbm.at[idx_v], rows_v, sem).wait()
            pltpu.sync_copy(rows_v, gs_hbm.at[pl.ds(base, C)])
            pltpu.sync_copy(dst_hbm.at[pl.ds(base, C)], idx_v)
            pltpu.async_copy(table_hbm.at[idx_v], rows_v, sem).wait()
            pltpu.sync_copy(rows_v, gd_hbm.at[pl.ds(base, C)])
            return carry

        lax.fori_loop(0, nchunks, body, 0)

    return k(table, src, dst)


def _sc_segmax(et, dst, C=2000):
    h, e = et.shape
    ew = e // NW
    nchunks = ew // C
    tw = h * NP
    nr = 8
    rw = tw // nr
    sl = rw // NS

    @functools.partial(
        pl.kernel, mesh=_MESH,
        out_type=jax.ShapeDtypeStruct((NC, tw), F32),
        scratch_types=[
            pltpu.VMEM((tw,), F32),
            pltpu.VMEM((C,), jnp.int32),
            pltpu.VMEM((h, C), F32),
            pltpu.VMEM((sl,), F32),
            pltpu.VMEM((sl,), F32),
            pltpu.VMEM_SHARED((NS, rw), F32),
        ],
        compiler_params=_SC_PARAMS_NL,
    )
    def k(et_hbm, dst_hbm, mp_hbm, loc, dstv, etv, ta, tb, shared):
        c = lax.axis_index("c")
        s = lax.axis_index("s")
        wid = s * NC + c
        neg = jnp.full((16,), -1e30, dtype=F32)

        def init(i, carry):
            loc[pl.ds(i * 16, 16)] = neg
            return carry
        lax.fori_loop(0, tw // 16, init, 0)

        def chunk(i, carry):
            base = wid * ew + i * C
            pltpu.sync_copy(dst_hbm.at[pl.ds(base, C)], dstv)
            pltpu.sync_copy(et_hbm.at[:, pl.ds(base, C)], etv)

            def grp(g, carry2):
                dv = dstv[pl.ds(g * 16, 16)]
                for j in range(h):
                    ev = etv[j, pl.ds(g * 16, 16)]
                    idx = dv + j * NP
                    cur = plsc.load_gather(loc, [idx])
                    plsc.store_scatter(loc, [idx], jnp.maximum(cur, ev))
                return carry2
            lax.fori_loop(0, C // 16, grp, 0)
            return carry
        lax.fori_loop(0, nchunks, chunk, 0)

        for r in range(nr):
            pltpu.sync_copy(loc.at[pl.ds(r * rw, rw)], shared.at[s])
            plsc.subcore_barrier()
            pltpu.sync_copy(shared.at[0, pl.ds(s * sl, sl)], ta)

            def merge(t, carry):
                pltpu.sync_copy(shared.at[t, pl.ds(s * sl, sl)], tb)

                def mv(v, carry2):
                    ta[pl.ds(v * 16, 16)] = jnp.maximum(
                        ta[pl.ds(v * 16, 16)], tb[pl.ds(v * 16, 16)])
                    return carry2
                lax.fori_loop(0, sl // 16, mv, 0)
                return carry
            lax.fori_loop(1, NS, merge, 0)
            pltpu.sync_copy(ta, mp_hbm.at[c, pl.ds(r * rw + s * sl, sl)])
            plsc.subcore_barrier()

    return k(et, dst)


def _sc_eepass(et, dst, mpart, z8, C=2000):
    h, e = et.shape
    ew = e // NW
    nchunks = ew // C
    tw = h * NP
    sl = tw // NS
    sn = NP // NS

    @functools.partial(
        pl.kernel, mesh=_MESH,
        out_type=[jax.ShapeDtypeStruct((e, h), F32),
                  jax.ShapeDtypeStruct((NC, NP, h), F32)],
        scratch_types=[
            pltpu.VMEM((tw,), F32),
            pltpu.VMEM((sl,), F32),
            pltpu.VMEM((C,), jnp.int32),
            pltpu.VMEM((h, C), F32),
            pltpu.VMEM((C, h), F32),
            pltpu.VMEM_SHARED((NP, h), F32),
        ],
        compiler_params=_SC_PARAMS_NL,
    )
    def k(et_hbm, dst_hbm, mp_hbm, z8_hbm, ee_hbm, dp_hbm,
          mloc, tb, dstv, etv, eev, acc_d):
        c = lax.axis_index("c")
        s = lax.axis_index("s")
        wid = s * NC + c
        lanes = lax.iota(jnp.int32, 16)
        pltpu.sync_copy(z8_hbm.at[pl.ds(s * sn, sn)], acc_d.at[pl.ds(s * sn, sn)])

        def stage(t, carry):
            pltpu.sync_copy(mp_hbm.at[0, pl.ds(t * sl, sl)],
                            mloc.at[pl.ds(t * sl, sl)])
            pltpu.sync_copy(mp_hbm.at[1, pl.ds(t * sl, sl)], tb)

            def mv(v, carry2):
                mloc[pl.ds(t * sl + v * 16, 16)] = jnp.maximum(
                    mloc[pl.ds(t * sl + v * 16, 16)], tb[pl.ds(v * 16, 16)])
                return carry2
            lax.fori_loop(0, sl // 16, mv, 0)
            return carry
        lax.fori_loop(0, NS, stage, 0)
        plsc.subcore_barrier()

        def chunk(i, carry):
            base = wid * ew + i * C
            pltpu.sync_copy(dst_hbm.at[pl.ds(base, C)], dstv)
            pltpu.sync_copy(et_hbm.at[:, pl.ds(base, C)], etv)

            def grp(g, carry2):
                dv = dstv[pl.ds(g * 16, 16)]
                rows = g * 16 + lanes
                for j in range(h):
                    idx = dv + j * NP
                    mv2 = plsc.load_gather(mloc, [idx])
                    ee = jnp.exp(etv[j, pl.ds(g * 16, 16)] - mv2)
                    cols = jnp.full((16,), j, dtype=jnp.int32)
                    plsc.store_scatter(eev, [rows, cols], ee)
                return carry2
            lax.fori_loop(0, C // 16, grp, 0)
            pltpu.sync_copy(eev, ee_hbm.at[pl.ds(base, C)])
            pltpu.sync_copy(eev, acc_d.at[dstv], add=True)
            return carry
        lax.fori_loop(0, nchunks, chunk, 0)

        plsc.subcore_barrier()
        pltpu.sync_copy(acc_d.at[pl.ds(s * sn, sn)], dp_hbm.at[c, pl.ds(s * sn, sn)])

    return k(et, dst, mpart, z8)


def _sc_scatter1(rows, dst, zrows, C=1000):
    e, f = rows.shape
    ew = e // NW
    nchunks = ew // C
    sl = NP // NS

    @functools.partial(
        pl.kernel, mesh=_MESH,
        out_type=jax.ShapeDtypeStruct((NC, NP, f), F32),
        scratch_types=[
            pltpu.VMEM((C,), jnp.int32),
            pltpu.VMEM((C, f), F32),
            pltpu.VMEM_SHARED((NP, f), F32),
        ],
        compiler_params=_SC_PARAMS,
    )
    def k(r_hbm, dst_hbm, z_hbm, op_hbm, idx_v, rv, acc):
        c = lax.axis_index("c")
        s = lax.axis_index("s")
        wid = s * NC + c
        pltpu.sync_copy(z_hbm.at[pl.ds(s * sl, sl)], acc.at[pl.ds(s * sl, sl)])
        plsc.subcore_barrier()

        def body(i, carry):
            base = wid * ew + i * C
            pltpu.sync_copy(dst_hbm.at[pl.ds(base, C)], idx_v)
            pltpu.sync_copy(r_hbm.at[pl.ds(base, C)], rv)
            pltpu.sync_copy(rv, acc.at[idx_v], add=True)
            return carry

        lax.fori_loop(0, nchunks, body, 0)
        plsc.subcore_barrier()
        pltpu.sync_copy(acc.at[pl.ds(s * sl, sl)], op_hbm.at[c, pl.ds(s * sl, sl)])

    return k(rows, dst, zrows)


def _sc_gather_d2(dpart, dst, C=2000):
    _, n, h = dpart.shape
    e = dst.shape[0]
    ew = e // NW
    nchunks = ew // C

    @functools.partial(
        pl.kernel, mesh=_MESH,
        out_type=[jax.ShapeDtypeStruct((e, h), F32)] * 2,
        scratch_types=[
            pltpu.VMEM((C,), jnp.int32),
            pltpu.VMEM((C, h), F32),
            pltpu.SemaphoreType.DMA,
        ],
        compiler_params=_SC_PARAMS,
    )
    def k(dp_hbm, dst_hbm, g0_hbm, g1_hbm, idx_v, rv, sem):
        wid = lax.axis_index("s") * NC + lax.axis_index("c")

        def body(i, carry):
            base = wid * ew + i * C
            pltpu.sync_copy(dst_hbm.at[pl.ds(base, C)], idx_v)
            pltpu.async_copy(dp_hbm.at[0].at[idx_v], rv, sem).wait()
            pltpu.sync_copy(rv, g0_hbm.at[pl.ds(base, C)])
            pltpu.async_copy(dp_hbm.at[1].at[idx_v], rv, sem).wait()
            pltpu.sync_copy(rv, g1_hbm.at[pl.ds(base, C)])
            return carry

        lax.fori_loop(0, nchunks, body, 0)

    return k(dpart, dst)


def _sc_take_out(opart, b2, idxpad):
    _, n, f = opart.shape
    q = idxpad.shape[0]
    qw = q // NW

    @functools.partial(
        pl.kernel, mesh=_MESH,
        out_type=jax.ShapeDtypeStruct((q, f), F32),
        scratch_types=[
            pltpu.VMEM((qw,), jnp.int32),
            pltpu.VMEM((qw, f), F32),
            pltpu.VMEM((qw, f), F32),
            pltpu.VMEM((f,), F32),
            pltpu.SemaphoreType.DMA,
        ],
        compiler_params=_SC_PARAMS,
    )
    def k(op_hbm, b2_hbm, idx_hbm, out_hbm, idx_v, r0, r1, bv, sem):
        wid = lax.axis_index("s") * NC + lax.axis_index("c")
        base = wid * qw
        pltpu.sync_copy(b2_hbm, bv)
        pltpu.sync_copy(idx_hbm.at[pl.ds(base, qw)], idx_v)
        pltpu.async_copy(op_hbm.at[0].at[idx_v], r0, sem).wait()
        pltpu.async_copy(op_hbm.at[1].at[idx_v], r1, sem).wait()
        b = bv[...]

        def rowadd(r, carry):
            r0[r, :] = r0[r, :] + r1[r, :] + b
            return carry
        lax.fori_loop(0, qw, rowadd, 0)
        pltpu.sync_copy(r0, out_hbm.at[pl.ds(base, qw)])

    return k(opart, b2, idxpad)


def kernel(x, edge_index, indices, W1, a1, b1, W2, a2, b2):
    n = x.shape[0]
    h1c, u1 = a1.shape
    h2c, u2 = a2.shape
    f1 = h1c * u1
    f2 = h2c * u2
    src = edge_index[0]
    dst = edge_index[1]

    eye1 = jnp.eye(h1c, dtype=F32)
    ab1 = (eye1[:, None, :] * a1[:, :, None]).reshape(f1, h1c)
    eh1 = jnp.repeat(eye1, u1, axis=1)
    eye2 = jnp.eye(h2c, dtype=F32)
    ab2 = (eye2[:, None, :] * a2[:, :, None]).reshape(f2, h2c)
    eh2 = jnp.repeat(eye2, u2, axis=1)
    mavg = jnp.tile(jnp.eye(u2, dtype=F32), (h2c, 1)) / h2c

    zf1 = jnp.zeros((NP, f1), F32)
    z8 = jnp.zeros((NP, h1c), F32)
    zo = jnp.zeros((NP, u2), F32)

    h1 = _matmul(x, W1)
    gs1, gd1 = _sc_gather2(h1, src, dst)
    et1 = _edge_eT(gs1, gd1, ab1)
    mp1 = _sc_segmax(et1, dst)
    ee1, dp1 = _sc_eepass(et1, dst, mp1, z8)
    w1 = _apply1(ee1, gs1, eh1)
    op1 = _sc_scatter1(w1, dst, zf1)
    h2 = _norm_mm(op1[0], op1[1], dp1[0], dp1[1], eh1, W2,
                  b1.reshape(1, f1))

    gs2, gd2 = _sc_gather2(h2, src, dst)
    et2 = _edge_eT(gs2, gd2, ab2)
    mp2 = _sc_segmax(et2, dst)
    ee2, dp2 = _sc_eepass(et2, dst, mp2, z8)
    d0g, d1g = _sc_gather_d2(dp2, dst)
    wred = _apply2(ee2, d0g, d1g, gs2, eh2, mavg)
    opr = _sc_scatter1(wred, dst, zo, C=2000)

    q = indices.shape[0]
    qpad = NW * ((q + NW - 1) // NW)
    idxpad = jnp.concatenate(
        [indices, jnp.zeros((qpad - q,), jnp.int32)])
    out = _sc_take_out(opr, b2, idxpad)
    return out[:q]
